# baseline (speedup 1.0000x reference)
import functools

import jax
import jax.numpy as jnp
from jax.experimental import pallas as pl
from jax.experimental.pallas import tpu as pltpu

K = 32


def _encode_body(x_ref, we_ref, be_ref, mean_ref, std_ref, bdec_ref, h_ref):
    xc = (x_ref[...] - mean_ref[...]) / std_ref[...] - bdec_ref[...]
    h = jax.lax.dot_general(
        xc, we_ref[...], (((1,), (1,)), ((), ())),
        preferred_element_type=jnp.float32)
    h_ref[...] = h + be_ref[...]


def _bitsearch_kth(m, k, axes):
    cnt0 = jnp.sum((m >= 0).astype(jnp.int32), axis=axes, keepdims=True)
    t0 = jnp.where(cnt0 >= k, jnp.int32(0), jnp.int32(-2147483648))

    def bit_body(b, t):
        cand = t + (jnp.int32(1) << (30 - b))
        cnt = jnp.sum((m >= cand).astype(jnp.int32), axis=axes, keepdims=True)
        return jnp.where(cnt >= k, cand, t)

    return jax.lax.fori_loop(0, 31, bit_body, t0)


def _select_decode_body(h_ref, wd_ref, bdec_ref, hs_ref, xhat_ref, m_ref):
    h = h_ref[...]
    BR, N = h.shape
    i = jax.lax.bitcast_convert_type(h, jnp.int32)
    m = jnp.where(i >= 0, i, i ^ jnp.int32(0x7FFFFFFF))
    m_ref[...] = m

    NL = 128
    NEG = jnp.int32(-2147483648)
    RG = 32
    group_tops = []
    for g in range(BR // RG):
        def col_body(c, lv, _g=g):
            v = m_ref[pl.ds(_g * RG, RG), pl.ds(c * NL, NL)]
            out = []
            for i in range(8):
                hi = jnp.maximum(lv[i], v)
                v = jnp.minimum(lv[i], v)
                out.append(hi)
            return tuple(out)

        levels = tuple(jnp.full((RG, NL), NEG, jnp.int32) for _ in range(8))
        levels = jax.lax.fori_loop(0, N // NL, col_body, levels)
        group_tops.append(jnp.stack(levels, axis=1))
    C = jnp.concatenate(group_tops, axis=0)

    Ct = jnp.transpose(C.reshape(BR, 8 * NL), (1, 0))
    t_cand = jnp.transpose(_bitsearch_kth(Ct, K, (0,)), (1, 0))
    cnt_full = jnp.sum((m >= t_cand).astype(jnp.int32), axis=1, keepdims=True)
    ok = jnp.all(cnt_full >= K)

    t = jax.lax.cond(ok, lambda: t_cand,
                     lambda: _bitsearch_kth(m, K, (1,)))
    hs = jnp.where(m >= t, h, 0.0)
    hs_ref[...] = hs
    xhat = jax.lax.dot_general(
        hs, wd_ref[...], (((1,), (1,)), ((), ())),
        preferred_element_type=jnp.float32)
    xhat_ref[...] = xhat + bdec_ref[...]


def _topk_sae(x, W_enc, b_enc, W_dec, b_dec, input_mean, input_std,
              interpret=False):
    B, D = x.shape
    N = W_enc.shape[0]
    BN = 512
    BR = 64

    be2 = b_enc.reshape(1, N)
    bd2 = b_dec.reshape(1, D)
    mean2 = input_mean.reshape(1, D)
    std2 = input_std.reshape(1, D)

    h_pre = pl.pallas_call(
        _encode_body,
        grid=(N // BN,),
        in_specs=[
            pl.BlockSpec((B, D), lambda j: (0, 0)),
            pl.BlockSpec((BN, D), lambda j: (j, 0)),
            pl.BlockSpec((1, BN), lambda j: (0, j)),
            pl.BlockSpec((1, D), lambda j: (0, 0)),
            pl.BlockSpec((1, D), lambda j: (0, 0)),
            pl.BlockSpec((1, D), lambda j: (0, 0)),
        ],
        out_specs=pl.BlockSpec((B, BN), lambda j: (0, j)),
        out_shape=jax.ShapeDtypeStruct((B, N), jnp.float32),
        compiler_params=pltpu.CompilerParams(
            dimension_semantics=("parallel",)),
        interpret=interpret,
    )(x, W_enc, be2, mean2, std2, bd2)

    h_sparse, x_hat = pl.pallas_call(
        _select_decode_body,
        grid=(B // BR,),
        in_specs=[
            pl.BlockSpec((BR, N), lambda i: (i, 0)),
            pl.BlockSpec((D, N), lambda i: (0, 0)),
            pl.BlockSpec((1, D), lambda i: (0, 0)),
        ],
        out_specs=[
            pl.BlockSpec((BR, N), lambda i: (i, 0)),
            pl.BlockSpec((BR, D), lambda i: (i, 0)),
        ],
        out_shape=[
            jax.ShapeDtypeStruct((B, N), jnp.float32),
            jax.ShapeDtypeStruct((B, D), jnp.float32),
        ],
        scratch_shapes=[pltpu.VMEM((BR, N), jnp.int32)],
        compiler_params=pltpu.CompilerParams(
            dimension_semantics=("parallel",)),
        interpret=interpret,
    )(h_pre, W_dec, bd2)

    return (x_hat, h_sparse, h_pre)


def kernel(x, W_enc, b_enc, W_dec, b_dec, input_mean, input_std):
    return _topk_sae(x, W_enc, b_enc, W_dec, b_dec, input_mean, input_std)


# speedup vs baseline: 1.3192x; 1.3192x over previous
import jax
import jax.numpy as jnp
from jax.experimental import pallas as pl
from jax.experimental.pallas import tpu as pltpu

K = 32
NL = 128
NC = 8


def _key(x):
    i = jax.lax.bitcast_convert_type(x, jnp.int32)
    return jnp.where(i >= 0, i, i ^ jnp.int32(0x7FFFFFFF))


def _bitsearch_kth(m, k, axes):
    cnt0 = jnp.sum((m >= 0).astype(jnp.int32), axis=axes, keepdims=True)
    t0 = jnp.where(cnt0 >= k, jnp.int32(0), jnp.int32(-2147483648))

    def bit_body(b, t):
        cand = t + (jnp.int32(1) << (30 - b))
        cnt = jnp.sum((m >= cand).astype(jnp.int32), axis=axes, keepdims=True)
        return jnp.where(cnt >= k, cand, t)

    return jax.lax.fori_loop(0, 31, bit_body, t0)


def _encode_fold_body(x_ref, we_ref, be_ref, mean_ref, std_ref, bdec_ref,
                      h_ref, c_ref):
    j = pl.program_id(0)
    xc = (x_ref[...] - mean_ref[...]) / std_ref[...] - bdec_ref[...]
    h = jax.lax.dot_general(
        xc, we_ref[...], (((1,), (1,)), ((), ())),
        preferred_element_type=jnp.float32)
    h = h + be_ref[...]
    h_ref[...] = h

    B, BN = h.shape

    @pl.when(j == 0)
    def _init():
        c_ref[...] = jnp.full(c_ref.shape, -jnp.inf, jnp.float32)

    for s in range(BN // NL):
        v = jax.lax.slice(h, (0, s * NL), (B, (s + 1) * NL))
        for i in range(NC):
            cur = c_ref[i]
            c_ref[i] = jnp.maximum(cur, v)
            v = jnp.minimum(cur, v)


def _threshold_body(c_ref, t_ref):
    Cv = c_ref[...]
    NC_, B, NL_ = Cv.shape
    Ck = _key(Cv)
    Ct = jnp.transpose(Ck, (0, 2, 1)).reshape(NC_ * NL_, B)
    t = _bitsearch_kth(Ct, K, (0,))
    t_ref[...] = jnp.broadcast_to(jnp.transpose(t, (1, 0)), t_ref.shape)


def _select_decode_body(h_ref, t_ref, wd_ref, bdec_ref, hs_ref, xhat_ref):
    h = h_ref[...]
    BR, N = h.shape
    m = _key(h)
    t_cand = t_ref[...][:, 0:1]

    maskf = (m >= t_cand).astype(jnp.float32)
    cnt = jnp.sum(maskf, axis=1, keepdims=True)
    ok = jnp.all(cnt >= K)

    def good():
        return maskf

    def bad():
        t = _bitsearch_kth(m, K, (1,))
        return (m >= t).astype(jnp.float32)

    mf = jax.lax.cond(ok, good, bad)
    hs = h * mf
    hs_ref[...] = hs
    xhat = jax.lax.dot_general(
        hs, wd_ref[...], (((1,), (1,)), ((), ())),
        preferred_element_type=jnp.float32)
    xhat_ref[...] = xhat + bdec_ref[...]


def _topk_sae(x, W_enc, b_enc, W_dec, b_dec, input_mean, input_std,
              interpret=False):
    B, D = x.shape
    N = W_enc.shape[0]
    BN = 512
    BR = 64

    be2 = b_enc.reshape(1, N)
    bd2 = b_dec.reshape(1, D)
    mean2 = input_mean.reshape(1, D)
    std2 = input_std.reshape(1, D)

    h_pre, C = pl.pallas_call(
        _encode_fold_body,
        grid=(N // BN,),
        in_specs=[
            pl.BlockSpec((B, D), lambda j: (0, 0)),
            pl.BlockSpec((BN, D), lambda j: (j, 0)),
            pl.BlockSpec((1, BN), lambda j: (0, j)),
            pl.BlockSpec((1, D), lambda j: (0, 0)),
            pl.BlockSpec((1, D), lambda j: (0, 0)),
            pl.BlockSpec((1, D), lambda j: (0, 0)),
        ],
        out_specs=[
            pl.BlockSpec((B, BN), lambda j: (0, j)),
            pl.BlockSpec((NC, B, NL), lambda j: (0, 0, 0)),
        ],
        out_shape=[
            jax.ShapeDtypeStruct((B, N), jnp.float32),
            jax.ShapeDtypeStruct((NC, B, NL), jnp.float32),
        ],
        compiler_params=pltpu.CompilerParams(
            dimension_semantics=("arbitrary",)),
        interpret=interpret,
    )(x, W_enc, be2, mean2, std2, bd2)

    t_bcast = pl.pallas_call(
        _threshold_body,
        out_shape=jax.ShapeDtypeStruct((B, NL), jnp.int32),
        interpret=interpret,
    )(C)

    h_sparse, x_hat = pl.pallas_call(
        _select_decode_body,
        grid=(B // BR,),
        in_specs=[
            pl.BlockSpec((BR, N), lambda i: (i, 0)),
            pl.BlockSpec((BR, NL), lambda i: (i, 0)),
            pl.BlockSpec((D, N), lambda i: (0, 0)),
            pl.BlockSpec((1, D), lambda i: (0, 0)),
        ],
        out_specs=[
            pl.BlockSpec((BR, N), lambda i: (i, 0)),
            pl.BlockSpec((BR, D), lambda i: (i, 0)),
        ],
        out_shape=[
            jax.ShapeDtypeStruct((B, N), jnp.float32),
            jax.ShapeDtypeStruct((B, D), jnp.float32),
        ],
        compiler_params=pltpu.CompilerParams(
            dimension_semantics=("parallel",)),
        interpret=interpret,
    )(h_pre, t_bcast, W_dec, bd2)

    return (x_hat, h_sparse, h_pre)


def kernel(x, W_enc, b_enc, W_dec, b_dec, input_mean, input_std):
    return _topk_sae(x, W_enc, b_enc, W_dec, b_dec, input_mean, input_std)


# speedup vs baseline: 1.8763x; 1.4223x over previous
import jax
import jax.numpy as jnp
from jax.experimental import pallas as pl
from jax.experimental.pallas import tpu as pltpu

K = 32
NL = 128
NC = 8


def _key(x):
    i = jax.lax.bitcast_convert_type(x, jnp.int32)
    return jnp.where(i >= 0, i, i ^ jnp.int32(0x7FFFFFFF))


def _bitsearch_kth(m, k, axes):
    cnt0 = jnp.sum((m >= 0).astype(jnp.int32), axis=axes, keepdims=True)
    t0 = jnp.where(cnt0 >= k, jnp.int32(0), jnp.int32(-2147483648))

    def bit_body(b, t):
        cand = t + (jnp.int32(1) << (30 - b))
        cnt = jnp.sum((m >= cand).astype(jnp.int32), axis=axes, keepdims=True)
        return jnp.where(cnt >= k, cand, t)

    return jax.lax.fori_loop(0, 31, bit_body, t0)


def _encode_fold_body(x_ref, we_ref, be_ref, mean_ref, std_ref, bdec_ref,
                      h_ref, c_ref):
    j = pl.program_id(0)
    xc = (x_ref[...] - mean_ref[...]) / std_ref[...] - bdec_ref[...]
    h = jax.lax.dot_general(
        xc, we_ref[...], (((1,), (1,)), ((), ())),
        preferred_element_type=jnp.float32)
    h = h + be_ref[...]
    h_ref[...] = h

    B, BN = h.shape

    @pl.when(j == 0)
    def _init():
        c_ref[...] = jnp.full(c_ref.shape, -jnp.inf, jnp.float32)

    def _ce(a, b):
        return jnp.maximum(a, b), jnp.minimum(a, b)

    RB = min(1024, B)
    for r in range(B // RB):
        r0 = r * RB
        v0, v1, v2, v3 = (
            jax.lax.slice(h, (r0, s * NL), (r0 + RB, (s + 1) * NL))
            for s in range(BN // NL))
        v0, v1 = _ce(v0, v1)
        v2, v3 = _ce(v2, v3)
        v0, v2 = _ce(v0, v2)
        v1, v3 = _ce(v1, v3)
        v1, v2 = _ce(v1, v2)

        s = [c_ref[i, pl.ds(r0, RB), :] for i in range(NC)]
        s[4] = jnp.maximum(s[4], v3)
        s[5] = jnp.maximum(s[5], v2)
        s[6] = jnp.maximum(s[6], v1)
        s[7] = jnp.maximum(s[7], v0)
        for d in (4, 2, 1):
            for i in range(NC):
                if (i % (2 * d)) < d:
                    s[i], s[i + d] = _ce(s[i], s[i + d])
        for i in range(NC):
            c_ref[i, pl.ds(r0, RB), :] = s[i]


def _threshold_body(c_ref, t_ref):
    Cv = c_ref[...]
    NC_, B, NL_ = Cv.shape
    Ck = _key(Cv)
    Ct = jnp.transpose(Ck, (0, 2, 1)).reshape(NC_ * NL_, B)
    t = _bitsearch_kth(Ct, K, (0,))
    t_ref[...] = jnp.broadcast_to(jnp.transpose(t, (1, 0)), t_ref.shape)


def _select_decode_body(h_ref, t_ref, wd_ref, bdec_ref, hs_ref, xhat_ref):
    h = h_ref[...]
    BR, N = h.shape
    m = _key(h)
    t_cand = t_ref[...][:, 0:1]

    maskf = (m >= t_cand).astype(jnp.float32)
    cnt = jnp.sum(maskf, axis=1, keepdims=True)
    ok = jnp.all(cnt >= K)

    def good():
        return maskf

    def bad():
        t = _bitsearch_kth(m, K, (1,))
        return (m >= t).astype(jnp.float32)

    mf = jax.lax.cond(ok, good, bad)
    hs = h * mf
    hs_ref[...] = hs
    xhat = jax.lax.dot_general(
        hs, wd_ref[...], (((1,), (1,)), ((), ())),
        preferred_element_type=jnp.float32)
    xhat_ref[...] = xhat + bdec_ref[...]


def _topk_sae(x, W_enc, b_enc, W_dec, b_dec, input_mean, input_std,
              interpret=False):
    B, D = x.shape
    N = W_enc.shape[0]
    BN = 512
    BR = 64

    be2 = b_enc.reshape(1, N)
    bd2 = b_dec.reshape(1, D)
    mean2 = input_mean.reshape(1, D)
    std2 = input_std.reshape(1, D)

    h_pre, C = pl.pallas_call(
        _encode_fold_body,
        grid=(N // BN,),
        in_specs=[
            pl.BlockSpec((B, D), lambda j: (0, 0)),
            pl.BlockSpec((BN, D), lambda j: (j, 0)),
            pl.BlockSpec((1, BN), lambda j: (0, j)),
            pl.BlockSpec((1, D), lambda j: (0, 0)),
            pl.BlockSpec((1, D), lambda j: (0, 0)),
            pl.BlockSpec((1, D), lambda j: (0, 0)),
        ],
        out_specs=[
            pl.BlockSpec((B, BN), lambda j: (0, j)),
            pl.BlockSpec((NC, B, NL), lambda j: (0, 0, 0)),
        ],
        out_shape=[
            jax.ShapeDtypeStruct((B, N), jnp.float32),
            jax.ShapeDtypeStruct((NC, B, NL), jnp.float32),
        ],
        compiler_params=pltpu.CompilerParams(
            dimension_semantics=("arbitrary",)),
        interpret=interpret,
    )(x, W_enc, be2, mean2, std2, bd2)

    t_bcast = pl.pallas_call(
        _threshold_body,
        out_shape=jax.ShapeDtypeStruct((B, NL), jnp.int32),
        interpret=interpret,
    )(C)

    h_sparse, x_hat = pl.pallas_call(
        _select_decode_body,
        grid=(B // BR,),
        in_specs=[
            pl.BlockSpec((BR, N), lambda i: (i, 0)),
            pl.BlockSpec((BR, NL), lambda i: (i, 0)),
            pl.BlockSpec((D, N), lambda i: (0, 0)),
            pl.BlockSpec((1, D), lambda i: (0, 0)),
        ],
        out_specs=[
            pl.BlockSpec((BR, N), lambda i: (i, 0)),
            pl.BlockSpec((BR, D), lambda i: (i, 0)),
        ],
        out_shape=[
            jax.ShapeDtypeStruct((B, N), jnp.float32),
            jax.ShapeDtypeStruct((B, D), jnp.float32),
        ],
        compiler_params=pltpu.CompilerParams(
            dimension_semantics=("parallel",)),
        interpret=interpret,
    )(h_pre, t_bcast, W_dec, bd2)

    return (x_hat, h_sparse, h_pre)


def kernel(x, W_enc, b_enc, W_dec, b_dec, input_mean, input_std):
    return _topk_sae(x, W_enc, b_enc, W_dec, b_dec, input_mean, input_std)


# speedup vs baseline: 2.0945x; 1.1163x over previous
import jax
import jax.numpy as jnp
from jax.experimental import pallas as pl
from jax.experimental.pallas import tpu as pltpu

K = 32
NL = 128
NC = 8


def _key(x):
    i = jax.lax.bitcast_convert_type(x, jnp.int32)
    return jnp.where(i >= 0, i, i ^ jnp.int32(0x7FFFFFFF))


def _bitsearch_kth(m, k, axes):
    cnt0 = jnp.sum((m >= 0).astype(jnp.int32), axis=axes, keepdims=True)
    t0 = jnp.where(cnt0 >= k, jnp.int32(0), jnp.int32(-2147483648))

    def bit_body(b, t):
        cand = t + (jnp.int32(1) << (30 - b))
        cnt = jnp.sum((m >= cand).astype(jnp.int32), axis=axes, keepdims=True)
        return jnp.where(cnt >= k, cand, t)

    return jax.lax.fori_loop(0, 31, bit_body, t0)


def _encode_fold_body(x_ref, we_ref, be_ref, mean_ref, std_ref, bdec_ref,
                      h_ref, c_ref):
    j = pl.program_id(0)
    xc = (x_ref[...] - mean_ref[...]) / std_ref[...] - bdec_ref[...]
    h = jax.lax.dot_general(
        xc, we_ref[...], (((1,), (1,)), ((), ())),
        preferred_element_type=jnp.float32)
    h = h + be_ref[...]
    h_ref[...] = h

    B, BN = h.shape

    @pl.when(j == 0)
    def _init():
        c_ref[...] = jnp.full(c_ref.shape, -jnp.inf, jnp.float32)

    def _ce(a, b):
        return jnp.maximum(a, b), jnp.minimum(a, b)

    RB = min(32, B)
    for r in range(B // RB):
        r0 = r * RB
        v0, v1, v2, v3 = (
            jax.lax.slice(h, (r0, s * NL), (r0 + RB, (s + 1) * NL))
            for s in range(BN // NL))
        v0, v1 = _ce(v0, v1)
        v2, v3 = _ce(v2, v3)
        v0, v2 = _ce(v0, v2)
        v1, v3 = _ce(v1, v3)
        v1, v2 = _ce(v1, v2)

        s = [c_ref[i, pl.ds(r0, RB), :] for i in range(NC)]
        s[4] = jnp.maximum(s[4], v3)
        s[5] = jnp.maximum(s[5], v2)
        s[6] = jnp.maximum(s[6], v1)
        s[7] = jnp.maximum(s[7], v0)
        for d in (4, 2, 1):
            for i in range(NC):
                if (i % (2 * d)) < d:
                    s[i], s[i + d] = _ce(s[i], s[i + d])
        for i in range(NC):
            c_ref[i, pl.ds(r0, RB), :] = s[i]


def _threshold_body(c_ref, t_ref):
    Cv = c_ref[...]
    NC_, B, NL_ = Cv.shape
    Ck = _key(Cv)
    Ct = jnp.transpose(Ck, (0, 2, 1)).reshape(NC_ * NL_, B)
    t = _bitsearch_kth(Ct, K, (0,))
    t_ref[...] = jnp.broadcast_to(jnp.transpose(t, (1, 0)), t_ref.shape)


def _select_decode_body(h_ref, t_ref, wd_ref, bdec_ref, hs_ref, xhat_ref):
    h = h_ref[...]
    BR, N = h.shape
    m = _key(h)
    t_cand = t_ref[...][:, 0:1]

    maskf = (m >= t_cand).astype(jnp.float32)
    cnt = jnp.sum(maskf, axis=1, keepdims=True)
    ok = jnp.all(cnt >= K)

    def good():
        return maskf

    def bad():
        t = _bitsearch_kth(m, K, (1,))
        return (m >= t).astype(jnp.float32)

    mf = jax.lax.cond(ok, good, bad)
    hs = h * mf
    hs_ref[...] = hs
    xhat = jax.lax.dot_general(
        hs, wd_ref[...], (((1,), (1,)), ((), ())),
        preferred_element_type=jnp.float32)
    xhat_ref[...] = xhat + bdec_ref[...]


def _topk_sae(x, W_enc, b_enc, W_dec, b_dec, input_mean, input_std,
              interpret=False):
    B, D = x.shape
    N = W_enc.shape[0]
    BN = 512
    BR = 64

    be2 = b_enc.reshape(1, N)
    bd2 = b_dec.reshape(1, D)
    mean2 = input_mean.reshape(1, D)
    std2 = input_std.reshape(1, D)

    h_pre, C = pl.pallas_call(
        _encode_fold_body,
        grid=(N // BN,),
        in_specs=[
            pl.BlockSpec((B, D), lambda j: (0, 0)),
            pl.BlockSpec((BN, D), lambda j: (j, 0)),
            pl.BlockSpec((1, BN), lambda j: (0, j)),
            pl.BlockSpec((1, D), lambda j: (0, 0)),
            pl.BlockSpec((1, D), lambda j: (0, 0)),
            pl.BlockSpec((1, D), lambda j: (0, 0)),
        ],
        out_specs=[
            pl.BlockSpec((B, BN), lambda j: (0, j)),
            pl.BlockSpec((NC, B, NL), lambda j: (0, 0, 0)),
        ],
        out_shape=[
            jax.ShapeDtypeStruct((B, N), jnp.float32),
            jax.ShapeDtypeStruct((NC, B, NL), jnp.float32),
        ],
        compiler_params=pltpu.CompilerParams(
            dimension_semantics=("arbitrary",)),
        interpret=interpret,
    )(x, W_enc, be2, mean2, std2, bd2)

    t_bcast = pl.pallas_call(
        _threshold_body,
        out_shape=jax.ShapeDtypeStruct((B, NL), jnp.int32),
        interpret=interpret,
    )(C)

    h_sparse, x_hat = pl.pallas_call(
        _select_decode_body,
        grid=(B // BR,),
        in_specs=[
            pl.BlockSpec((BR, N), lambda i: (i, 0)),
            pl.BlockSpec((BR, NL), lambda i: (i, 0)),
            pl.BlockSpec((D, N), lambda i: (0, 0)),
            pl.BlockSpec((1, D), lambda i: (0, 0)),
        ],
        out_specs=[
            pl.BlockSpec((BR, N), lambda i: (i, 0)),
            pl.BlockSpec((BR, D), lambda i: (i, 0)),
        ],
        out_shape=[
            jax.ShapeDtypeStruct((B, N), jnp.float32),
            jax.ShapeDtypeStruct((B, D), jnp.float32),
        ],
        compiler_params=pltpu.CompilerParams(
            dimension_semantics=("parallel",)),
        interpret=interpret,
    )(h_pre, t_bcast, W_dec, bd2)

    return (x_hat, h_sparse, h_pre)


def kernel(x, W_enc, b_enc, W_dec, b_dec, input_mean, input_std):
    return _topk_sae(x, W_enc, b_enc, W_dec, b_dec, input_mean, input_std)


# speedup vs baseline: 2.2064x; 1.0535x over previous
import jax
import jax.numpy as jnp
from jax.experimental import pallas as pl
from jax.experimental.pallas import tpu as pltpu

K = 32
NL = 128
NC = 8


def _key(x):
    i = jax.lax.bitcast_convert_type(x, jnp.int32)
    return jnp.where(i >= 0, i, i ^ jnp.int32(0x7FFFFFFF))


def _bitsearch_kth(m, k, axes):
    cnt0 = jnp.sum((m >= 0).astype(jnp.int32), axis=axes, keepdims=True)
    t0 = jnp.where(cnt0 >= k, jnp.int32(0), jnp.int32(-2147483648))

    def bit_body(b, t):
        cand = t + (jnp.int32(1) << (30 - b))
        cnt = jnp.sum((m >= cand).astype(jnp.int32), axis=axes, keepdims=True)
        return jnp.where(cnt >= k, cand, t)

    return jax.lax.fori_loop(0, 31, bit_body, t0)


def _encode_fold_body(x_ref, we_ref, be_ref, mean_ref, std_ref, bdec_ref,
                      h_ref, c_ref):
    j = pl.program_id(0)
    xc = (x_ref[...] - mean_ref[...]) / std_ref[...] - bdec_ref[...]
    h = jax.lax.dot_general(
        xc, we_ref[...], (((1,), (1,)), ((), ())),
        preferred_element_type=jnp.float32)
    h = h + be_ref[...]
    h_ref[...] = h

    B, BN = h.shape

    @pl.when(j == 0)
    def _init():
        c_ref[...] = jnp.full(c_ref.shape, -jnp.inf, jnp.float32)

    def _ce(a, b):
        return jnp.maximum(a, b), jnp.minimum(a, b)

    RB = min(32, B)
    for r in range(B // RB):
        r0 = r * RB
        v0, v1, v2, v3 = (
            jax.lax.slice(h, (r0, s * NL), (r0 + RB, (s + 1) * NL))
            for s in range(BN // NL))
        v0, v1 = _ce(v0, v1)
        v2, v3 = _ce(v2, v3)
        v0, v2 = _ce(v0, v2)
        v1, v3 = _ce(v1, v3)
        v1, v2 = _ce(v1, v2)

        s = [c_ref[i, pl.ds(r0, RB), :] for i in range(NC)]
        s[4] = jnp.maximum(s[4], v3)
        s[5] = jnp.maximum(s[5], v2)
        s[6] = jnp.maximum(s[6], v1)
        s[7] = jnp.maximum(s[7], v0)
        for d in (4, 2, 1):
            for i in range(NC):
                if (i % (2 * d)) < d:
                    s[i], s[i + d] = _ce(s[i], s[i + d])
        for i in range(NC):
            c_ref[i, pl.ds(r0, RB), :] = s[i]


def _threshold_body(c_ref, t_ref):
    Cv = c_ref[...]
    NC_, B, NL_ = Cv.shape
    Ck = _key(Cv)
    Ct = jnp.transpose(Ck, (0, 2, 1)).reshape(NC_ * NL_, B)
    t = _bitsearch_kth(Ct, K, (0,))
    tb = jnp.where(t >= 0, t, t ^ jnp.int32(0x7FFFFFFF))
    tf = jax.lax.bitcast_convert_type(tb, jnp.float32)
    t_ref[...] = jnp.broadcast_to(jnp.transpose(tf, (1, 0)), t_ref.shape)


def _select_decode_body(h_ref, t_ref, wd_ref, bdec_ref, hs_ref, xhat_ref):
    h = h_ref[...]
    BR, N = h.shape
    t_cand = t_ref[...][:, 0:1]

    maskf = (h >= t_cand).astype(jnp.float32)
    cnt = jnp.sum(maskf, axis=1, keepdims=True)
    ok = jnp.all(cnt >= K)

    def good():
        return maskf

    def bad():
        m = _key(h)
        t = _bitsearch_kth(m, K, (1,))
        return (m >= t).astype(jnp.float32)

    mf = jax.lax.cond(ok, good, bad)
    hs = h * mf
    hs_ref[...] = hs
    xhat = jax.lax.dot_general(
        hs, wd_ref[...], (((1,), (1,)), ((), ())),
        preferred_element_type=jnp.float32)
    xhat_ref[...] = xhat + bdec_ref[...]


def _topk_sae(x, W_enc, b_enc, W_dec, b_dec, input_mean, input_std,
              interpret=False):
    B, D = x.shape
    N = W_enc.shape[0]
    BN = 512
    BR = 64

    be2 = b_enc.reshape(1, N)
    bd2 = b_dec.reshape(1, D)
    mean2 = input_mean.reshape(1, D)
    std2 = input_std.reshape(1, D)

    h_pre, C = pl.pallas_call(
        _encode_fold_body,
        grid=(N // BN,),
        in_specs=[
            pl.BlockSpec((B, D), lambda j: (0, 0)),
            pl.BlockSpec((BN, D), lambda j: (j, 0)),
            pl.BlockSpec((1, BN), lambda j: (0, j)),
            pl.BlockSpec((1, D), lambda j: (0, 0)),
            pl.BlockSpec((1, D), lambda j: (0, 0)),
            pl.BlockSpec((1, D), lambda j: (0, 0)),
        ],
        out_specs=[
            pl.BlockSpec((B, BN), lambda j: (0, j)),
            pl.BlockSpec((NC, B, NL), lambda j: (0, 0, 0)),
        ],
        out_shape=[
            jax.ShapeDtypeStruct((B, N), jnp.float32),
            jax.ShapeDtypeStruct((NC, B, NL), jnp.float32),
        ],
        compiler_params=pltpu.CompilerParams(
            dimension_semantics=("arbitrary",)),
        interpret=interpret,
    )(x, W_enc, be2, mean2, std2, bd2)

    t_bcast = pl.pallas_call(
        _threshold_body,
        out_shape=jax.ShapeDtypeStruct((B, NL), jnp.float32),
        interpret=interpret,
    )(C)

    h_sparse, x_hat = pl.pallas_call(
        _select_decode_body,
        grid=(B // BR,),
        in_specs=[
            pl.BlockSpec((BR, N), lambda i: (i, 0)),
            pl.BlockSpec((BR, NL), lambda i: (i, 0)),
            pl.BlockSpec((D, N), lambda i: (0, 0)),
            pl.BlockSpec((1, D), lambda i: (0, 0)),
        ],
        out_specs=[
            pl.BlockSpec((BR, N), lambda i: (i, 0)),
            pl.BlockSpec((BR, D), lambda i: (i, 0)),
        ],
        out_shape=[
            jax.ShapeDtypeStruct((B, N), jnp.float32),
            jax.ShapeDtypeStruct((B, D), jnp.float32),
        ],
        compiler_params=pltpu.CompilerParams(
            dimension_semantics=("parallel",)),
        interpret=interpret,
    )(h_pre, t_bcast, W_dec, bd2)

    return (x_hat, h_sparse, h_pre)


def kernel(x, W_enc, b_enc, W_dec, b_dec, input_mean, input_std):
    return _topk_sae(x, W_enc, b_enc, W_dec, b_dec, input_mean, input_std)
